# Initial kernel scaffold; baseline (speedup 1.0000x reference)
import jax
import jax.numpy as jnp
from jax.experimental import pallas as pl


def kernel(x, edge_index, W1, W2, lin1_w, lin1_b, g1, b1, rm1, rv1, lin2_w, lin2_b, g2, b2, rm2, rv2):
    raise NotImplementedError("write your pallas kernel here")


# speedup vs baseline: 32.0733x; 32.0733x over previous
import functools

import jax
import jax.numpy as jnp
from jax import lax
from jax.experimental import pallas as pl
from jax.experimental.pallas import tpu as pltpu
from jax.experimental.pallas import tpu_sc as plsc

_EPS = 1e-5
_LAMDA = 0.001
_NC, _NS = 2, 16
_NW = _NC * _NS
_CHUNK = 128
_ACC = 10240
_ZR = _ACC // _NS
_BLK = 2000

def _mesh():
    return plsc.VectorSubcoreMesh(core_axis_name="c", subcore_axis_name="s",
                                  num_cores=_NC, num_subcores=_NS)


def _make_deg_kernel(kch):
    @functools.partial(
        pl.kernel,
        out_type=jax.ShapeDtypeStruct((_NC, _ACC), jnp.float32),
        mesh=_mesh(),
        scratch_types=[
            pltpu.VMEM((kch, _CHUNK), jnp.int32),
            pltpu.VMEM((_CHUNK,), jnp.float32),
            pltpu.VMEM((_ZR,), jnp.float32),
            pltpu.VMEM_SHARED((_ACC,), jnp.float32),
        ],
    )
    def deg_kernel(dst_hbm, degp_hbm, idx_v, ones_v, zer_v, acc_sh):
        c = lax.axis_index("c")
        s = lax.axis_index("s")
        wid = c * _NS + s

        def zrow(i, carry):
            zer_v[pl.ds(i * 16, 16)] = jnp.zeros((16,), jnp.float32)
            return carry

        lax.fori_loop(0, _ZR // 16, zrow, 0)
        for i in range(_CHUNK // 16):
            ones_v[pl.ds(i * 16, 16)] = jnp.ones((16,), jnp.float32)
        pltpu.sync_copy(zer_v, acc_sh.at[pl.ds(s * _ZR, _ZR)])
        pltpu.sync_copy(dst_hbm.at[wid], idx_v)
        plsc.subcore_barrier()

        def body(j, carry):
            pltpu.sync_copy(ones_v, acc_sh.at[idx_v.at[j]], add=True)
            return carry

        lax.fori_loop(0, kch, body, 0)
        plsc.subcore_barrier()

        @pl.when(s == 0)
        def _dump():
            pltpu.sync_copy(acc_sh, degp_hbm.at[c])

    return deg_kernel


def _make_agg_kernel(kch, n, f):
    @functools.partial(
        pl.kernel,
        out_type=jax.ShapeDtypeStruct((_NC, _ACC, f), jnp.float32),
        mesh=_mesh(),
        scratch_types=[
            pltpu.VMEM((kch, _CHUNK), jnp.int32),
            pltpu.VMEM((kch, _CHUNK), jnp.int32),
            pltpu.VMEM((_CHUNK, f), jnp.float32),
            pltpu.VMEM((_ZR, f), jnp.float32),
            pltpu.VMEM_SHARED((_ACC, f), jnp.float32),
            pltpu.SemaphoreType.DMA,
        ],
        compiler_params=pltpu.CompilerParams(use_tc_tiling_on_sc=False),
    )
    def agg_kernel(h_hbm, src_hbm, dst_hbm, pp_hbm, sidx, didx, rows, zer,
                   acc_sh, sem):
        c = lax.axis_index("c")
        s = lax.axis_index("s")
        wid = c * _NS + s

        def zrow(i, carry):
            zer[i, :] = jnp.zeros((f,), jnp.float32)
            return carry

        lax.fori_loop(0, _ZR, zrow, 0)
        pltpu.sync_copy(zer, acc_sh.at[pl.ds(s * _ZR, _ZR)])
        pltpu.sync_copy(src_hbm.at[wid], sidx)
        pltpu.sync_copy(dst_hbm.at[wid], didx)
        plsc.subcore_barrier()

        def body(j, carry):
            pltpu.async_copy(h_hbm.at[sidx.at[j]], rows, sem).wait()
            pltpu.sync_copy(rows, acc_sh.at[didx.at[j]], add=True)
            return carry

        lax.fori_loop(0, kch, body, 0)
        plsc.subcore_barrier()

        @pl.when(s == 0)
        def _dump():
            pltpu.sync_copy(acc_sh, pp_hbm.at[c])

    return agg_kernel


def _tc1_body(x_ref, w_ref, degt_ref, hs_ref, dinv_ref):
    h = jnp.dot(x_ref[...], w_ref[...], preferred_element_type=jnp.float32)
    deg = degt_ref[:, 0:1] + degt_ref[:, 1:2] + 1.0
    dinv = lax.rsqrt(deg)
    hs_ref[...] = h * dinv
    dinv_ref[...] = jnp.broadcast_to(dinv, dinv_ref.shape)


def _tc1(x, w1, degt, n, f):
    grid = n // _BLK
    return pl.pallas_call(
        _tc1_body,
        grid=(grid,),
        in_specs=[
            pl.BlockSpec((_BLK, x.shape[1]), lambda i: (i, 0)),
            pl.BlockSpec(w1.shape, lambda i: (0, 0)),
            pl.BlockSpec((_BLK, 2), lambda i: (i, 0)),
        ],
        out_specs=[
            pl.BlockSpec((_BLK, f), lambda i: (i, 0)),
            pl.BlockSpec((_BLK, f), lambda i: (i, 0)),
        ],
        out_shape=[
            jax.ShapeDtypeStruct((n, f), jnp.float32),
            jax.ShapeDtypeStruct((n, f), jnp.float32),
        ],
    )(x, w1, degt)


def _tc2_body(p0_ref, p1_ref, hs_ref, dinv_ref, lw_ref, lb_ref, a_ref, c_ref,
              w2_ref, out_ref):
    agg = dinv_ref[...] * (p0_ref[...] + p1_ref[...] + hs_ref[...])
    z = jnp.dot(agg, lw_ref[...], preferred_element_type=jnp.float32)
    s = jax.nn.softmax(z + lb_ref[...], axis=-1)
    t = jnp.dot(s, a_ref[...], preferred_element_type=jnp.float32)
    d = agg + _LAMDA * (agg * t + c_ref[...])
    r = jnp.maximum(d, 0.0)
    h2 = jnp.dot(r, w2_ref[...], preferred_element_type=jnp.float32)
    out_ref[...] = h2 * dinv_ref[...]


def _tc3_body(p0_ref, p1_ref, hs_ref, dinv_ref, lw_ref, lb_ref, a_ref, c_ref,
              out_ref):
    agg = dinv_ref[...] * (p0_ref[...] + p1_ref[...] + hs_ref[...])
    z = jnp.dot(agg, lw_ref[...], preferred_element_type=jnp.float32)
    s = jax.nn.softmax(z + lb_ref[...], axis=-1)
    t = jnp.dot(s, a_ref[...], preferred_element_type=jnp.float32)
    d = agg + _LAMDA * (agg * t + c_ref[...])
    out_ref[...] = jnp.maximum(d, 0.0)


def _tc_norm(body, p0, p1, hs, dinv16, lw, lb, a, cvec, *extra):
    n, f = hs.shape
    grid = n // _BLK
    row = lambda i: (i, 0)
    zero = lambda i: (0, 0)
    in_specs = [
        pl.BlockSpec((_BLK, f), row),
        pl.BlockSpec((_BLK, f), row),
        pl.BlockSpec((_BLK, f), row),
        pl.BlockSpec((_BLK, f), row),
        pl.BlockSpec(lw.shape, zero),
        pl.BlockSpec(lb.shape, zero),
        pl.BlockSpec(a.shape, zero),
        pl.BlockSpec(cvec.shape, zero),
    ] + [pl.BlockSpec(e.shape, zero) for e in extra]
    return pl.pallas_call(
        body,
        grid=(grid,),
        in_specs=in_specs,
        out_specs=pl.BlockSpec((_BLK, f), row),
        out_shape=jax.ShapeDtypeStruct((n, f), jnp.float32),
    )(p0, p1, hs, dinv16, lw, lb, a, cvec, *extra)


def kernel(x, edge_index, W1, W2, lin1_w, lin1_b, g1, b1, rm1, rv1,
           lin2_w, lin2_b, g2, b2, rm2, rv2):
    n, d_in = x.shape
    hid = W1.shape[1]
    c_out = W2.shape[1]
    g = lin1_w.shape[1]
    e = edge_index.shape[1]

    kch = -(-e // (_NW * _CHUNK))
    e_pad = _NW * kch * _CHUNK
    pad = e_pad - e
    src = jnp.concatenate([edge_index[0], jnp.zeros((pad,), jnp.int32)])
    dst = jnp.concatenate([edge_index[1], jnp.full((pad,), n, jnp.int32)])
    srcp = src.reshape(_NW, kch, _CHUNK)
    dstp = dst.reshape(_NW, kch, _CHUNK)

    a1 = (g1 * lax.rsqrt(rv1 + _EPS))
    c1 = (b1 - rm1 * a1).reshape(g, hid).sum(0, keepdims=True)
    a1 = a1.reshape(g, hid)
    a2f = (g2 * lax.rsqrt(rv2 + _EPS))
    c2 = (b2 - rm2 * a2f).reshape(g, c_out).sum(0, keepdims=True)
    a2 = a2f.reshape(g, c_out)
    fpad = hid - c_out
    a2p = jnp.pad(a2, ((0, 0), (0, fpad)))
    c2p = jnp.pad(c2, ((0, 0), (0, fpad)))
    w2p = jnp.pad(W2, ((0, 0), (0, fpad)))
    l2wp = jnp.pad(lin2_w, ((0, fpad), (0, 0)))
    l1b = lin1_b.reshape(1, g)
    l2b = lin2_b.reshape(1, g)

    deg_kernel = _make_deg_kernel(kch)
    agg_kernel = _make_agg_kernel(kch, n, hid)

    degp = deg_kernel(dstp)
    degt = degp[:, :n].T

    h0s, dinv16 = _tc1(x, W1, degt, n, hid)

    pp = agg_kernel(h0s, srcp, dstp)
    h2s = _tc_norm(_tc2_body, pp[0, :n], pp[1, :n], h0s, dinv16,
                   lin1_w, l1b, a1, c1, w2p)

    qq = agg_kernel(h2s, srcp, dstp)
    o16 = _tc_norm(_tc3_body, qq[0, :n], qq[1, :n], h2s, dinv16,
                   l2wp, l2b, a2p, c2p)
    return o16[:, :c_out]
